# baseline (speedup 1.0000x reference)
import functools

import jax
import jax.numpy as jnp
from jax import lax
from jax.experimental import pallas as pl
from jax.experimental.pallas import tpu as pltpu
from jax.experimental.pallas import tpu_sc as plsc

BATCH = 16384
HIST = 50
EMBED_DIM = 64
NUM_IDX = BATCH * HIST

NC, NS = 2, 16
NW = NC * NS
B_PER_W = NUM_IDX // NW
BB = 128
NBSUB = BATCH // NW // BB
UNITS = NBSUB * HIST


@jax.jit
def _sc_gather(x_flat, table):
    mesh = plsc.VectorSubcoreMesh(
        core_axis_name="c", subcore_axis_name="s", num_cores=NC, num_subcores=NS
    )

    @functools.partial(
        pl.kernel,
        out_type=jax.ShapeDtypeStruct(
            (HIST, EMBED_DIM // 8, BATCH // BB, 8, BB), jnp.float32
        ),
        mesh=mesh,
        scratch_types=[
            pltpu.VMEM((B_PER_W,), jnp.int32),
            pltpu.VMEM((2, BB), jnp.int32),
            pltpu.VMEM((2, BB, EMBED_DIM), jnp.float32),
            pltpu.VMEM((2, 8, 8, BB), jnp.float32),
            pltpu.SemaphoreType.DMA,
            pltpu.SemaphoreType.DMA,
            pltpu.SemaphoreType.DMA,
            pltpu.SemaphoreType.DMA,
        ],
        compiler_params=pltpu.CompilerParams(
            use_tc_tiling_on_sc=False, needs_layout_passes=False
        ),
    )
    def k(idx_hbm, table_hbm, out_hbm, idx_v, gidx, rows, obuf, g0, g1, o0, o1):
        gsem = [g0, g1]
        osem = [o0, o1]
        wid = lax.axis_index("s") * NC + lax.axis_index("c")
        base = wid * B_PER_W
        pltpu.sync_copy(idx_hbm.at[pl.ds(base, B_PER_W)], idx_v)

        iota = lax.broadcasted_iota(jnp.int32, (16,), 0)
        iota50 = iota * HIST
        dt_vec = [(iota + 16 * kk) // 8 for kk in range(4)]
        di_vec = [(iota + 16 * kk) - dt_vec[kk] * 8 for kk in range(4)]

        def build_gidx(u, b):
            bsub = u // HIST
            h = u - bsub * HIST
            off = bsub * (BB * HIST) + h
            for kk in range(8):
                vals = plsc.load_gather(idx_v, [iota50 + (off + kk * 16 * HIST)])
                gidx[b, pl.ds(kk * 16, 16)] = vals

        def g_start(b):
            pltpu.async_copy(table_hbm.at[gidx.at[b]], rows.at[b], gsem[b])

        def g_wait(b):
            pltpu.make_async_copy(
                table_hbm.at[pl.ds(0, BB)], rows.at[b], gsem[b]
            ).wait()

        def transpose(b):
            @plsc.parallel_loop(0, BB, 1, unroll=8)
            def jbody(j):
                jsplat = jnp.zeros((16,), jnp.int32) + j
                vals = [rows[b, j, pl.ds(kk * 16, 16)] for kk in range(4)]
                for kk in range(4):
                    plsc.store_scatter(
                        obuf.at[b], [dt_vec[kk], di_vec[kk], jsplat], vals[kk]
                    )

        def o_start(u, b):
            bsub = u // HIST
            h = u - bsub * HIST
            btg = wid * NBSUB + bsub
            pltpu.async_copy(obuf.at[b], out_hbm.at[h, :, btg], osem[b])

        def o_wait(b):
            pltpu.make_async_copy(
                rows.at[b], table_hbm.at[pl.ds(0, BB)], osem[b]
            ).wait()

        build_gidx(0, 0)
        g_start(0)

        @pl.loop(0, UNITS, step=2)
        def _(i):
            for b in (0, 1):
                u = i + b
                un = u + 1

                @pl.when(un < UNITS)
                def _():
                    build_gidx(un, 1 - b)
                    g_start(1 - b)

                g_wait(b)

                @pl.when(u >= 2)
                def _():
                    o_wait(b)

                transpose(b)
                o_start(u, b)

        o_wait(0)
        o_wait(1)

    out5 = k(x_flat, table)
    return out5.transpose((2, 4, 0, 1, 3)).reshape(BATCH, HIST, EMBED_DIM)


def kernel(x, table):
    return _sc_gather(x.reshape(-1), table)


# speedup vs baseline: 1.0949x; 1.0949x over previous
import functools

import jax
import jax.numpy as jnp
from jax import lax
from jax.experimental import pallas as pl
from jax.experimental.pallas import tpu as pltpu
from jax.experimental.pallas import tpu_sc as plsc

BATCH = 16384
HIST = 50
EMBED_DIM = 64
NUM_IDX = BATCH * HIST

NC, NS = 2, 16
NW = NC * NS
B_PER_W = NUM_IDX // NW
BB = 128
NBSUB = BATCH // NW // BB
UNITS = NBSUB * HIST


@jax.jit
def _sc_gather(x_flat, table):
    mesh = plsc.VectorSubcoreMesh(
        core_axis_name="c", subcore_axis_name="s", num_cores=NC, num_subcores=NS
    )

    @functools.partial(
        pl.kernel,
        out_type=jax.ShapeDtypeStruct(
            (HIST, EMBED_DIM // 8, BATCH // BB, 8, BB), jnp.float32
        ),
        mesh=mesh,
        scratch_types=[
            pltpu.VMEM((B_PER_W,), jnp.int32),
            pltpu.VMEM((2, BB), jnp.int32),
            pltpu.VMEM((2, BB, EMBED_DIM), jnp.float32),
            pltpu.VMEM((2, 8, 8, BB), jnp.float32),
            pltpu.SemaphoreType.DMA,
            pltpu.SemaphoreType.DMA,
            pltpu.SemaphoreType.DMA,
            pltpu.SemaphoreType.DMA,
        ],
        compiler_params=pltpu.CompilerParams(
            use_tc_tiling_on_sc=False, needs_layout_passes=False
        ),
    )
    def k(idx_hbm, table_hbm, out_hbm, idx_v, gidx, rows, obuf, g0, g1, o0, o1):
        gsem = [g0, g1]
        osem = [o0, o1]
        wid = lax.axis_index("s") * NC + lax.axis_index("c")
        base = wid * B_PER_W
        pltpu.sync_copy(idx_hbm.at[pl.ds(base, B_PER_W)], idx_v)

        iota = lax.broadcasted_iota(jnp.int32, (16,), 0)
        iota50 = iota * HIST
        jv = [iota + 16 * kk for kk in range(8)]

        def build_gidx(u, b):
            bsub = u // HIST
            h = u - bsub * HIST
            off = bsub * (BB * HIST) + h
            for kk in range(8):
                vals = plsc.load_gather(idx_v, [iota50 + (off + kk * 16 * HIST)])
                gidx[b, pl.ds(kk * 16, 16)] = vals

        def g_start(b):
            pltpu.async_copy(table_hbm.at[gidx.at[b]], rows.at[b], gsem[b])

        def g_wait(b):
            pltpu.make_async_copy(
                table_hbm.at[pl.ds(0, BB)], rows.at[b], gsem[b]
            ).wait()

        def transpose(b):
            @plsc.parallel_loop(0, EMBED_DIM, 1, unroll=8)
            def dbody(d):
                dt = d // 8
                di = d - dt * 8
                dv = jnp.zeros((16,), jnp.int32) + d
                for kk in range(4):
                    v_a = plsc.load_gather(rows.at[b], [jv[2 * kk], dv])
                    v_b = plsc.load_gather(rows.at[b], [jv[2 * kk + 1], dv])
                    obuf[b, dt, di, pl.ds(2 * kk * 16, 16)] = v_a
                    obuf[b, dt, di, pl.ds((2 * kk + 1) * 16, 16)] = v_b

        def o_start(u, b):
            bsub = u // HIST
            h = u - bsub * HIST
            btg = wid * NBSUB + bsub
            pltpu.async_copy(obuf.at[b], out_hbm.at[h, :, btg], osem[b])

        def o_wait(b):
            pltpu.make_async_copy(
                rows.at[b], table_hbm.at[pl.ds(0, BB)], osem[b]
            ).wait()

        build_gidx(0, 0)
        g_start(0)

        @pl.loop(0, UNITS, step=2)
        def _(i):
            for b in (0, 1):
                u = i + b
                un = u + 1

                @pl.when(un < UNITS)
                def _():
                    build_gidx(un, 1 - b)
                    g_start(1 - b)

                g_wait(b)

                @pl.when(u >= 2)
                def _():
                    o_wait(b)

                transpose(b)
                o_start(u, b)

        o_wait(0)
        o_wait(1)

    out5 = k(x_flat, table)
    return out5.transpose((2, 4, 0, 1, 3)).reshape(BATCH, HIST, EMBED_DIM)


def kernel(x, table):
    return _sc_gather(x.reshape(-1), table)


# speedup vs baseline: 1.1832x; 1.0806x over previous
import functools

import jax
import jax.numpy as jnp
from jax import lax
from jax.experimental import pallas as pl
from jax.experimental.pallas import tpu as pltpu
from jax.experimental.pallas import tpu_sc as plsc

BATCH = 16384
HIST = 50
EMBED_DIM = 64
NUM_IDX = BATCH * HIST

NC, NS = 2, 16
NW = NC * NS
B_PER_W = NUM_IDX // NW
CHUNK = 320
NSTEPS = B_PER_W // CHUNK
NBUF = 4


@jax.jit
def _sc_gather(x_flat, table):
    mesh = plsc.VectorSubcoreMesh(
        core_axis_name="c", subcore_axis_name="s", num_cores=NC, num_subcores=NS
    )

    @functools.partial(
        pl.kernel,
        out_type=jax.ShapeDtypeStruct((NUM_IDX, EMBED_DIM), jnp.float32),
        mesh=mesh,
        scratch_types=[
            pltpu.VMEM((B_PER_W,), jnp.int32),
            pltpu.VMEM((NBUF, CHUNK, EMBED_DIM), jnp.float32),
        ]
        + [pltpu.SemaphoreType.DMA] * (2 * NBUF),
        compiler_params=pltpu.CompilerParams(use_tc_tiling_on_sc=False),
    )
    def k(idx_hbm, table_hbm, out_hbm, idx_v, rows_v, *sems):
        gsem, osem = sems[:NBUF], sems[NBUF:]
        wid = lax.axis_index("s") * NC + lax.axis_index("c")
        base = wid * B_PER_W
        pltpu.sync_copy(idx_hbm.at[pl.ds(base, B_PER_W)], idx_v)

        def g_start(j, b):
            pltpu.async_copy(
                table_hbm.at[idx_v.at[pl.ds(j * CHUNK, CHUNK)]],
                rows_v.at[b],
                gsem[b],
            )

        def g_wait(b):
            pltpu.make_async_copy(
                table_hbm.at[pl.ds(0, CHUNK)], rows_v.at[b], gsem[b]
            ).wait()

        def o_start(j, b):
            pltpu.async_copy(
                rows_v.at[b], out_hbm.at[pl.ds(base + j * CHUNK, CHUNK)], osem[b]
            )

        def o_wait(b):
            pltpu.make_async_copy(
                rows_v.at[b], out_hbm.at[pl.ds(base, CHUNK)], osem[b]
            ).wait()

        for b in range(NBUF - 1):
            g_start(b, b)

        @pl.loop(0, NSTEPS, step=NBUF)
        def _(i):
            for b in range(NBUF):
                j = i + b
                jg = j + (NBUF - 1)
                bg = (b + NBUF - 1) % NBUF

                @pl.when(jg < NSTEPS)
                def _():
                    @pl.when(jg >= NBUF)
                    def _():
                        o_wait(bg)

                    g_start(jg, bg)

                g_wait(b)
                o_start(j, b)

        for b in range(NBUF):
            o_wait(b)

    return k(x_flat, table)


def kernel(x, table):
    out = _sc_gather(x.reshape(-1), table)
    return out.reshape(BATCH, HIST, EMBED_DIM)
